# baseline (speedup 1.0000x reference)
import functools

import jax
import jax.numpy as jnp
from jax import lax
from jax.experimental import pallas as pl
from jax.experimental.pallas import tpu as pltpu
from jax.experimental.pallas import tpu_sc as plsc

D = 32
NC = 2
NS = 16
NW = NC * NS
CHUNK = 1600
NBUF = 2


@functools.cache
def _tc_detile_call(V: int):
    n_blk = (V + 511) // 512

    @functools.partial(
        pl.pallas_call,
        grid=(n_blk,),
        in_specs=[pl.BlockSpec((D, 512), lambda k: (0, k))],
        out_specs=pl.BlockSpec((128, 128), lambda k: (k, 0)),
        out_shape=jax.ShapeDtypeStruct((V * D // 128, 128), jnp.float32),
    )
    def tc_detile(in_ref, out_ref):
        t = in_ref[...].T
        r_iota = lax.broadcasted_iota(jnp.int32, (128, 512), 0)
        v_iota = lax.broadcasted_iota(jnp.int32, (128, 512), 1)
        d_iota = lax.broadcasted_iota(jnp.int32, (D, 128), 0)
        l_iota = lax.broadcasted_iota(jnp.int32, (D, 128), 1)
        acc = jnp.zeros((128, 128), jnp.float32)
        for h in range(4):
            sel = (v_iota == 4 * r_iota + h).astype(jnp.float32)
            put = (l_iota == 32 * h + d_iota).astype(jnp.float32)
            acc += jax.lax.dot(
                jax.lax.dot(sel, t, preferred_element_type=jnp.float32),
                put, preferred_element_type=jnp.float32)
        out_ref[...] = acc

    return tc_detile


@functools.cache
def _gather_call(B: int, V: int):
    b_per_w = B // NW
    nchunks = b_per_w // CHUNK
    mesh = plsc.VectorSubcoreMesh(core_axis_name="c", subcore_axis_name="s")

    @functools.partial(
        pl.kernel,
        mesh=mesh,
        out_type=jax.ShapeDtypeStruct((B, D), jnp.float32),
        scratch_types=[
            [pltpu.VMEM((CHUNK,), jnp.int32)] * NBUF,
            [pltpu.VMEM((CHUNK, D), jnp.float32)] * NBUF,
            [pltpu.SemaphoreType.DMA] * NBUF,
            [pltpu.SemaphoreType.DMA] * NBUF,
        ],
        compiler_params=pltpu.CompilerParams(use_tc_tiling_on_sc=False),
    )
    def gather(idx_hbm, table_hbm, out_hbm, idx_vs, rows_vs, gsems, osems):
        wid = lax.axis_index("s") * NC + lax.axis_index("c")
        base = wid * b_per_w

        gcopy = {}
        ocopy = {}

        def start(g):
            b = g % NBUF
            off = base + g * CHUNK
            pltpu.sync_copy(idx_hbm.at[pl.ds(off, CHUNK)], idx_vs[b])
            c = pltpu.make_async_copy(
                table_hbm.at[idx_vs[b]], rows_vs[b], gsems[b])
            c.start()
            gcopy[g] = c

        def drain(g):
            b = g % NBUF
            off = base + g * CHUNK
            gcopy[g].wait()
            c = pltpu.make_async_copy(
                rows_vs[b], out_hbm.at[pl.ds(off, CHUNK)], osems[b])
            c.start()
            ocopy[g] = c

        for g in range(nchunks):
            if g >= NBUF:
                ocopy[g - NBUF].wait()
            start(g)
            if g >= 1:
                drain(g - 1)
        drain(nchunks - 1)
        ocopy[nchunks - 2].wait()
        ocopy[nchunks - 1].wait()

    return gather


def kernel(indices, weight):
    BT, T = indices.shape
    B = BT * T
    V = weight.shape[0]
    w2 = _tc_detile_call(V)(weight.T)
    idx = indices.T.reshape(-1).astype(jnp.int32)
    out = _gather_call(B, V)(idx, w2.reshape(V, D))
    return out.reshape(T, BT, D).transpose(1, 0, 2)


# speedup vs baseline: 1.8535x; 1.8535x over previous
import functools

import jax
import jax.numpy as jnp
from jax import lax
from jax.experimental import pallas as pl
from jax.experimental.pallas import tpu as pltpu
from jax.experimental.pallas import tpu_sc as plsc

D = 32
NC = 2
NS = 16
NW = NC * NS
CHUNK = 1600
NBUF = 2
LANES = 16


@functools.cache
def _detile_call(V: int):
    n_full = V // 128
    rem = V - n_full * 128
    k_rr = n_full // NW
    n_tail = n_full - k_rr * NW
    mesh = plsc.VectorSubcoreMesh(core_axis_name="c", subcore_axis_name="s")

    BLK = D * 128

    @functools.partial(
        pl.kernel,
        mesh=mesh,
        out_type=jax.ShapeDtypeStruct((V * D,), jnp.float32),
        scratch_types=[
            [pltpu.VMEM((D, 128), jnp.float32)] * NBUF,
            [pltpu.VMEM((BLK,), jnp.float32)] * NBUF,
            pltpu.VMEM((max(rem, 1) * D,), jnp.float32),
            [pltpu.SemaphoreType.DMA] * NBUF,
            [pltpu.SemaphoreType.DMA] * NBUF,
        ],
        compiler_params=pltpu.CompilerParams(needs_layout_passes=False),
    )
    def detile(wt_hbm, tail_hbm, w2_hbm, in_vs, tr_vs, tail_v, isems, osems):
        wid = lax.axis_index("s") * NC + lax.axis_index("c")
        iota = lax.broadcasted_iota(jnp.int32, (LANES,), 0)
        iota_d = iota * D

        iota_dr = [iota_d + r for r in range(8)]

        def transpose_block(b):
            span = (LANES - 1) * D + 8
            LA = 12
            pairs = [(d, k) for d in range(D) for k in range(128 // LANES)]
            pending = []
            for n, (d, k) in enumerate(pairs):
                pending.append(
                    (d, k, in_vs[b][d, pl.ds(k * LANES, LANES)]))
                if n >= LA:
                    pd, pk, src = pending.pop(0)
                    plsc.store_scatter(
                        tr_vs[b].at[
                            pl.ds(pk * LANES * D + (pd // 8) * 8, span)],
                        [iota_dr[pd % 8]], src)
            for pd, pk, src in pending:
                plsc.store_scatter(
                    tr_vs[b].at[pl.ds(pk * LANES * D + (pd // 8) * 8, span)],
                    [iota_dr[pd % 8]], src)

        def start_in(b, c):
            pltpu.make_async_copy(
                wt_hbm.at[:, pl.ds(c * 128, 128)], in_vs[b], isems[b]).start()

        def wait_in(b):
            pltpu.make_async_copy(
                wt_hbm.at[:, pl.ds(0, 128)], in_vs[b], isems[b]).wait()

        def start_out(b, c):
            pltpu.make_async_copy(
                tr_vs[b], w2_hbm.at[pl.ds(c * BLK, BLK)], osems[b]).start()

        def wait_out(b):
            pltpu.make_async_copy(
                tr_vs[b], w2_hbm.at[pl.ds(0, BLK)], osems[b]).wait()

        def col(k):
            return wid + k * NW

        start_in(0, col(0))
        start_in(1, col(1))
        for b in range(NBUF):
            wait_in(b)
            transpose_block(b)
            start_in(b, col(b + NBUF))
            start_out(b, col(b))

        @pl.loop(0, (k_rr - NBUF) // NBUF)
        def _(i):
            for b in range(NBUF):
                k = NBUF + i * NBUF + b
                c = col(k)
                wait_out(b)
                wait_in(b)
                transpose_block(b)

                @pl.when(k + NBUF < k_rr)
                def _():
                    start_in(b, c + NBUF * NW)

                start_out(b, c)

        wait_out(0)
        wait_out(1)

        if n_tail:
            @pl.when(wid < n_tail)
            def _():
                c = k_rr * NW + wid
                pltpu.sync_copy(wt_hbm.at[:, pl.ds(c * 128, 128)], in_vs[0])
                transpose_block(0)
                pltpu.sync_copy(tr_vs[0], w2_hbm.at[pl.ds(c * BLK, BLK)])

        if rem:
            @pl.when(wid == n_tail)
            def _():
                pltpu.sync_copy(tail_hbm, tail_v)
                for j in range(rem):
                    jvec = jnp.full((LANES,), j, jnp.int32)
                    for half in range(D // LANES):
                        src = plsc.load_gather(
                            tail_v, [(half * LANES + iota) * rem + jvec])
                        tr_vs[1][pl.ds(D * j + half * LANES, LANES)] = src
                pltpu.sync_copy(
                    tr_vs[1].at[pl.ds(0, rem * D)],
                    w2_hbm.at[pl.ds(n_full * BLK, rem * D)])

    return detile


@functools.cache
def _gather_call(B: int, V: int):
    b_per_w = B // NW
    nchunks = b_per_w // CHUNK
    mesh = plsc.VectorSubcoreMesh(core_axis_name="c", subcore_axis_name="s")

    @functools.partial(
        pl.kernel,
        mesh=mesh,
        out_type=jax.ShapeDtypeStruct((B, D), jnp.float32),
        scratch_types=[
            [pltpu.VMEM((CHUNK,), jnp.int32)] * NBUF,
            [pltpu.VMEM((CHUNK, D), jnp.float32)] * NBUF,
            [pltpu.SemaphoreType.DMA] * NBUF,
            [pltpu.SemaphoreType.DMA] * NBUF,
        ],
        compiler_params=pltpu.CompilerParams(use_tc_tiling_on_sc=False),
    )
    def gather(idx_hbm, table_hbm, out_hbm, idx_vs, rows_vs, gsems, osems):
        wid = lax.axis_index("s") * NC + lax.axis_index("c")
        base = wid * b_per_w

        gcopy = {}
        ocopy = {}

        def start(g):
            b = g % NBUF
            off = base + g * CHUNK
            pltpu.sync_copy(idx_hbm.at[pl.ds(off, CHUNK)], idx_vs[b])
            c = pltpu.make_async_copy(
                table_hbm.at[idx_vs[b]], rows_vs[b], gsems[b])
            c.start()
            gcopy[g] = c

        def drain(g):
            b = g % NBUF
            off = base + g * CHUNK
            gcopy[g].wait()
            c = pltpu.make_async_copy(
                rows_vs[b], out_hbm.at[pl.ds(off, CHUNK)], osems[b])
            c.start()
            ocopy[g] = c

        for g in range(nchunks):
            if g >= NBUF:
                ocopy[g - NBUF].wait()
            start(g)
            if g >= 1:
                drain(g - 1)
        drain(nchunks - 1)
        ocopy[nchunks - 2].wait()
        ocopy[nchunks - 1].wait()

    return gather


def kernel(indices, weight):
    BT, T = indices.shape
    B = BT * T
    V = weight.shape[0]
    n_full = V // 128
    rem = V - n_full * 128
    tail = weight[n_full * 128:].T.reshape(-1) if rem else jnp.zeros(
        (D,), jnp.float32)
    w2 = _detile_call(V)(weight.T, tail)
    idx = indices.T.reshape(-1).astype(jnp.int32)
    out = _gather_call(B, V)(idx, w2.reshape(V, D))
    return out.reshape(T, BT, D).transpose(1, 0, 2)


# speedup vs baseline: 1.8544x; 1.0005x over previous
import functools

import jax
import jax.numpy as jnp
from jax import lax
from jax.experimental import pallas as pl
from jax.experimental.pallas import tpu as pltpu
from jax.experimental.pallas import tpu_sc as plsc

D = 32
NC = 2
NS = 16
NW = NC * NS
CHUNK = 1600
NBUF = 2
LANES = 16


@functools.cache
def _detile_call(V: int):
    n_full = V // 128
    rem = V - n_full * 128
    k_rr = n_full // NW
    n_tail = n_full - k_rr * NW
    mesh = plsc.VectorSubcoreMesh(core_axis_name="c", subcore_axis_name="s")

    BLK = D * 128

    @functools.partial(
        pl.kernel,
        mesh=mesh,
        out_type=jax.ShapeDtypeStruct((V * D,), jnp.float32),
        scratch_types=[
            [pltpu.VMEM((D, 128), jnp.float32)] * NBUF,
            [pltpu.VMEM((BLK,), jnp.float32)] * NBUF,
            pltpu.VMEM((max(rem, 1) * D,), jnp.float32),
            [pltpu.SemaphoreType.DMA] * NBUF,
            [pltpu.SemaphoreType.DMA] * NBUF,
        ],
        compiler_params=pltpu.CompilerParams(needs_layout_passes=False),
    )
    def detile(wt_hbm, tail_hbm, w2_hbm, in_vs, tr_vs, tail_v, isems, osems):
        wid = lax.axis_index("s") * NC + lax.axis_index("c")
        iota = lax.broadcasted_iota(jnp.int32, (LANES,), 0)
        iota_d = iota * D

        iota_dr = [iota_d + r for r in range(8)]

        def transpose_block(b):
            span = (LANES - 1) * D + 8
            LA = 12
            pairs = [(d, k) for d in range(D) for k in range(128 // LANES)]
            pending = []

            def flush_one():
                pd, pk, src = pending.pop(0)
                plsc.store_scatter(
                    tr_vs[b].at[pl.ds(pk * LANES * D + (pd // 8) * 8, span)],
                    [iota_dr[pd % 8]], src)

            for n, (d, k) in enumerate(pairs):
                pending.append(
                    (d, k, in_vs[b][d, pl.ds(k * LANES, LANES)]))
                if n >= LA:
                    flush_one()
            while pending:
                flush_one()

        def start_in(b, c):
            pltpu.make_async_copy(
                wt_hbm.at[:, pl.ds(c * 128, 128)], in_vs[b], isems[b]).start()

        def wait_in(b):
            pltpu.make_async_copy(
                wt_hbm.at[:, pl.ds(0, 128)], in_vs[b], isems[b]).wait()

        def start_out(b, c):
            pltpu.make_async_copy(
                tr_vs[b], w2_hbm.at[pl.ds(c * BLK, BLK)], osems[b]).start()

        def wait_out(b):
            pltpu.make_async_copy(
                tr_vs[b], w2_hbm.at[pl.ds(0, BLK)], osems[b]).wait()

        def col(k):
            return wid + k * NW

        start_in(0, col(0))
        start_in(1, col(1))
        for b in range(NBUF):
            wait_in(b)
            transpose_block(b)
            start_in(b, col(b + NBUF))
            start_out(b, col(b))

        @pl.loop(0, (k_rr - NBUF) // NBUF)
        def _(i):
            for b in range(NBUF):
                k = NBUF + i * NBUF + b
                c = col(k)
                wait_out(b)
                wait_in(b)
                transpose_block(b)

                @pl.when(k + NBUF < k_rr)
                def _():
                    start_in(b, c + NBUF * NW)

                start_out(b, c)

        wait_out(0)
        wait_out(1)

        if n_tail:
            @pl.when(wid < n_tail)
            def _():
                c = k_rr * NW + wid
                pltpu.sync_copy(wt_hbm.at[:, pl.ds(c * 128, 128)], in_vs[0])
                transpose_block(0)
                pltpu.sync_copy(tr_vs[0], w2_hbm.at[pl.ds(c * BLK, BLK)])

        if rem:
            @pl.when(wid == n_tail)
            def _():
                pltpu.sync_copy(tail_hbm, tail_v)
                for j in range(rem):
                    jvec = jnp.full((LANES,), j, jnp.int32)
                    for half in range(D // LANES):
                        src = plsc.load_gather(
                            tail_v, [(half * LANES + iota) * rem + jvec])
                        tr_vs[1][pl.ds(D * j + half * LANES, LANES)] = src
                pltpu.sync_copy(
                    tr_vs[1].at[pl.ds(0, rem * D)],
                    w2_hbm.at[pl.ds(n_full * BLK, rem * D)])

    return detile


@functools.cache
def _gather_call(B: int, V: int):
    b_per_w = B // NW
    nchunks = b_per_w // CHUNK
    mesh = plsc.VectorSubcoreMesh(core_axis_name="c", subcore_axis_name="s")

    @functools.partial(
        pl.kernel,
        mesh=mesh,
        out_type=jax.ShapeDtypeStruct((B, D), jnp.float32),
        scratch_types=[
            [pltpu.VMEM((CHUNK,), jnp.int32)] * NBUF,
            [pltpu.VMEM((CHUNK, D), jnp.float32)] * NBUF,
            [pltpu.SemaphoreType.DMA] * NBUF,
            [pltpu.SemaphoreType.DMA] * NBUF,
        ],
        compiler_params=pltpu.CompilerParams(use_tc_tiling_on_sc=False),
    )
    def gather(idx_hbm, table_hbm, out_hbm, idx_vs, rows_vs, gsems, osems):
        wid = lax.axis_index("s") * NC + lax.axis_index("c")
        base = wid * b_per_w

        gcopy = {}
        ocopy = {}

        def start(g):
            b = g % NBUF
            off = base + g * CHUNK
            pltpu.sync_copy(idx_hbm.at[pl.ds(off, CHUNK)], idx_vs[b])
            c = pltpu.make_async_copy(
                table_hbm.at[idx_vs[b]], rows_vs[b], gsems[b])
            c.start()
            gcopy[g] = c

        def drain(g):
            b = g % NBUF
            off = base + g * CHUNK
            gcopy[g].wait()
            c = pltpu.make_async_copy(
                rows_vs[b], out_hbm.at[pl.ds(off, CHUNK)], osems[b])
            c.start()
            ocopy[g] = c

        for g in range(nchunks):
            if g >= NBUF:
                ocopy[g - NBUF].wait()
            start(g)
            if g >= 1:
                drain(g - 1)
        drain(nchunks - 1)
        ocopy[nchunks - 2].wait()
        ocopy[nchunks - 1].wait()

    return gather


def kernel(indices, weight):
    BT, T = indices.shape
    B = BT * T
    V = weight.shape[0]
    n_full = V // 128
    rem = V - n_full * 128
    tail = weight[n_full * 128:].T.reshape(-1) if rem else jnp.zeros(
        (D,), jnp.float32)
    w2 = _detile_call(V)(weight.T, tail)
    idx = indices.T.reshape(-1).astype(jnp.int32)
    out = _gather_call(B, V)(idx, w2.reshape(V, D))
    return out.reshape(T, BT, D).transpose(1, 0, 2)


# speedup vs baseline: 2.1422x; 1.1552x over previous
import functools

import jax
import jax.numpy as jnp
from jax import lax
from jax.experimental import pallas as pl
from jax.experimental.pallas import tpu as pltpu
from jax.experimental.pallas import tpu_sc as plsc

D = 32
NC = 2
NS = 16
NW = NC * NS
CHUNK = 1600
NBUF = 2
LANES = 16


@functools.cache
def _detile_call(V: int):
    n_full = V // 128
    rem = V - n_full * 128
    k_rr = n_full // NW
    n_tail = n_full - k_rr * NW
    mesh = plsc.VectorSubcoreMesh(core_axis_name="c", subcore_axis_name="s")

    BLK = D * 128

    @functools.partial(
        pl.kernel,
        mesh=mesh,
        out_type=jax.ShapeDtypeStruct((V * D,), jnp.float32),
        scratch_types=[
            [pltpu.VMEM((D, 128), jnp.float32)] * NBUF,
            [pltpu.VMEM((BLK,), jnp.float32)] * NBUF,
            pltpu.VMEM((max(rem, 1) * D,), jnp.float32),
            [pltpu.SemaphoreType.DMA] * NBUF,
            [pltpu.SemaphoreType.DMA] * NBUF,
        ],
        compiler_params=pltpu.CompilerParams(needs_layout_passes=False),
    )
    def detile(wt_hbm, tail_hbm, w2_hbm, in_vs, tr_vs, tail_v, isems, osems):
        wid = lax.axis_index("s") * NC + lax.axis_index("c")
        iota = lax.broadcasted_iota(jnp.int32, (LANES,), 0)
        iota_d = iota * D

        iota_dr = [iota_d + r for r in range(8)]

        def transpose_block(b):
            span = (LANES - 1) * D + 8

            @plsc.parallel_loop(0, D, 1, unroll=4)
            def _(d):
                base8 = pl.multiple_of((d // 8) * 8, 8)
                idxv = iota_d + (d % 8)
                for k in range(128 // LANES):
                    src = in_vs[b][d, pl.ds(k * LANES, LANES)]
                    plsc.store_scatter(
                        tr_vs[b].at[pl.ds(k * LANES * D + base8, span)],
                        [idxv], src)

        def start_in(b, c):
            pltpu.make_async_copy(
                wt_hbm.at[:, pl.ds(c * 128, 128)], in_vs[b], isems[b]).start()

        def wait_in(b):
            pltpu.make_async_copy(
                wt_hbm.at[:, pl.ds(0, 128)], in_vs[b], isems[b]).wait()

        def start_out(b, c):
            pltpu.make_async_copy(
                tr_vs[b], w2_hbm.at[pl.ds(c * BLK, BLK)], osems[b]).start()

        def wait_out(b):
            pltpu.make_async_copy(
                tr_vs[b], w2_hbm.at[pl.ds(0, BLK)], osems[b]).wait()

        def col(k):
            return wid + k * NW

        start_in(0, col(0))
        start_in(1, col(1))
        for b in range(NBUF):
            wait_in(b)
            transpose_block(b)
            start_in(b, col(b + NBUF))
            start_out(b, col(b))

        @pl.loop(0, (k_rr - NBUF) // NBUF)
        def _(i):
            for b in range(NBUF):
                k = NBUF + i * NBUF + b
                c = col(k)
                wait_out(b)
                wait_in(b)
                transpose_block(b)

                @pl.when(k + NBUF < k_rr)
                def _():
                    start_in(b, c + NBUF * NW)

                start_out(b, c)

        wait_out(0)
        wait_out(1)

        if n_tail:
            @pl.when(wid < n_tail)
            def _():
                c = k_rr * NW + wid
                pltpu.sync_copy(wt_hbm.at[:, pl.ds(c * 128, 128)], in_vs[0])
                transpose_block(0)
                pltpu.sync_copy(tr_vs[0], w2_hbm.at[pl.ds(c * BLK, BLK)])

        if rem:
            @pl.when(wid == n_tail)
            def _():
                pltpu.sync_copy(tail_hbm, tail_v)
                for j in range(rem):
                    jvec = jnp.full((LANES,), j, jnp.int32)
                    for half in range(D // LANES):
                        src = plsc.load_gather(
                            tail_v, [(half * LANES + iota) * rem + jvec])
                        tr_vs[1][pl.ds(D * j + half * LANES, LANES)] = src
                pltpu.sync_copy(
                    tr_vs[1].at[pl.ds(0, rem * D)],
                    w2_hbm.at[pl.ds(n_full * BLK, rem * D)])

    return detile


@functools.cache
def _gather_call(B: int, V: int):
    b_per_w = B // NW
    nchunks = b_per_w // CHUNK
    mesh = plsc.VectorSubcoreMesh(core_axis_name="c", subcore_axis_name="s")

    @functools.partial(
        pl.kernel,
        mesh=mesh,
        out_type=jax.ShapeDtypeStruct((B, D), jnp.float32),
        scratch_types=[
            [pltpu.VMEM((CHUNK,), jnp.int32)] * NBUF,
            [pltpu.VMEM((CHUNK, D), jnp.float32)] * NBUF,
            [pltpu.SemaphoreType.DMA] * NBUF,
            [pltpu.SemaphoreType.DMA] * NBUF,
        ],
        compiler_params=pltpu.CompilerParams(use_tc_tiling_on_sc=False),
    )
    def gather(idx_hbm, table_hbm, out_hbm, idx_vs, rows_vs, gsems, osems):
        wid = lax.axis_index("s") * NC + lax.axis_index("c")
        base = wid * b_per_w

        gcopy = {}
        ocopy = {}

        def start(g):
            b = g % NBUF
            off = base + g * CHUNK
            pltpu.sync_copy(idx_hbm.at[pl.ds(off, CHUNK)], idx_vs[b])
            c = pltpu.make_async_copy(
                table_hbm.at[idx_vs[b]], rows_vs[b], gsems[b])
            c.start()
            gcopy[g] = c

        def drain(g):
            b = g % NBUF
            off = base + g * CHUNK
            gcopy[g].wait()
            c = pltpu.make_async_copy(
                rows_vs[b], out_hbm.at[pl.ds(off, CHUNK)], osems[b])
            c.start()
            ocopy[g] = c

        for g in range(nchunks):
            if g >= NBUF:
                ocopy[g - NBUF].wait()
            start(g)
            if g >= 1:
                drain(g - 1)
        drain(nchunks - 1)
        ocopy[nchunks - 2].wait()
        ocopy[nchunks - 1].wait()

    return gather


def kernel(indices, weight):
    BT, T = indices.shape
    B = BT * T
    V = weight.shape[0]
    n_full = V // 128
    rem = V - n_full * 128
    tail = weight[n_full * 128:].T.reshape(-1) if rem else jnp.zeros(
        (D,), jnp.float32)
    w2 = _detile_call(V)(weight.T, tail)
    idx = indices.T.reshape(-1).astype(jnp.int32)
    out = _gather_call(B, V)(idx, w2.reshape(V, D))
    return out.reshape(T, BT, D).transpose(1, 0, 2)


# speedup vs baseline: 2.1527x; 1.0049x over previous
import functools

import jax
import jax.numpy as jnp
from jax import lax
from jax.experimental import pallas as pl
from jax.experimental.pallas import tpu as pltpu
from jax.experimental.pallas import tpu_sc as plsc

D = 32
NC = 2
NS = 16
NW = NC * NS
CHUNK = 1600
NBUF = 2
LANES = 16


@functools.cache
def _detile_call(V: int):
    n_full = V // 128
    rem = V - n_full * 128
    k_rr = n_full // NW
    n_tail = n_full - k_rr * NW
    mesh = plsc.VectorSubcoreMesh(core_axis_name="c", subcore_axis_name="s")

    BLK = D * 128

    @functools.partial(
        pl.kernel,
        mesh=mesh,
        out_type=jax.ShapeDtypeStruct((V * D,), jnp.float32),
        scratch_types=[
            [pltpu.VMEM((D, 128), jnp.float32)] * NBUF,
            [pltpu.VMEM((BLK,), jnp.float32)] * NBUF,
            pltpu.VMEM((max(rem, 1) * D,), jnp.float32),
            [pltpu.SemaphoreType.DMA] * NBUF,
            [pltpu.SemaphoreType.DMA] * NBUF,
        ],
        compiler_params=pltpu.CompilerParams(needs_layout_passes=False),
    )
    def detile(wt_hbm, tail_hbm, w2_hbm, in_vs, tr_vs, tail_v, isems, osems):
        wid = lax.axis_index("s") * NC + lax.axis_index("c")
        iota = lax.broadcasted_iota(jnp.int32, (LANES,), 0)
        iota_d = iota * D

        iota_dr = [iota_d + r for r in range(8)]

        def transpose_block(b):
            span = (LANES - 1) * D + 8

            @plsc.parallel_loop(0, D, 1, unroll=8)
            def _(d):
                base8 = pl.multiple_of((d // 8) * 8, 8)
                idxv = iota_d + (d % 8)
                for k in range(128 // LANES):
                    src = in_vs[b][d, pl.ds(k * LANES, LANES)]
                    plsc.store_scatter(
                        tr_vs[b].at[pl.ds(k * LANES * D + base8, span)],
                        [idxv], src)

        def start_in(b, c):
            pltpu.make_async_copy(
                wt_hbm.at[:, pl.ds(c * 128, 128)], in_vs[b], isems[b]).start()

        def wait_in(b):
            pltpu.make_async_copy(
                wt_hbm.at[:, pl.ds(0, 128)], in_vs[b], isems[b]).wait()

        def start_out(b, c):
            pltpu.make_async_copy(
                tr_vs[b], w2_hbm.at[pl.ds(c * BLK, BLK)], osems[b]).start()

        def wait_out(b):
            pltpu.make_async_copy(
                tr_vs[b], w2_hbm.at[pl.ds(0, BLK)], osems[b]).wait()

        def col(k):
            return wid + k * NW

        start_in(0, col(0))
        start_in(1, col(1))
        for b in range(NBUF):
            wait_in(b)
            transpose_block(b)
            start_in(b, col(b + NBUF))
            start_out(b, col(b))

        @pl.loop(0, (k_rr - NBUF) // NBUF)
        def _(i):
            for b in range(NBUF):
                k = NBUF + i * NBUF + b
                c = col(k)
                wait_out(b)
                wait_in(b)
                transpose_block(b)

                @pl.when(k + NBUF < k_rr)
                def _():
                    start_in(b, c + NBUF * NW)

                start_out(b, c)

        wait_out(0)
        wait_out(1)

        if n_tail:
            @pl.when(wid < n_tail)
            def _():
                c = k_rr * NW + wid
                pltpu.sync_copy(wt_hbm.at[:, pl.ds(c * 128, 128)], in_vs[0])
                transpose_block(0)
                pltpu.sync_copy(tr_vs[0], w2_hbm.at[pl.ds(c * BLK, BLK)])

        if rem:
            @pl.when(wid == n_tail)
            def _():
                pltpu.sync_copy(tail_hbm, tail_v)
                for j in range(rem):
                    jvec = jnp.full((LANES,), j, jnp.int32)
                    for half in range(D // LANES):
                        src = plsc.load_gather(
                            tail_v, [(half * LANES + iota) * rem + jvec])
                        tr_vs[1][pl.ds(D * j + half * LANES, LANES)] = src
                pltpu.sync_copy(
                    tr_vs[1].at[pl.ds(0, rem * D)],
                    w2_hbm.at[pl.ds(n_full * BLK, rem * D)])

    return detile


@functools.cache
def _gather_call(B: int, V: int):
    b_per_w = B // NW
    nchunks = b_per_w // CHUNK
    mesh = plsc.VectorSubcoreMesh(core_axis_name="c", subcore_axis_name="s")

    @functools.partial(
        pl.kernel,
        mesh=mesh,
        out_type=jax.ShapeDtypeStruct((B, D), jnp.float32),
        scratch_types=[
            [pltpu.VMEM((CHUNK,), jnp.int32)] * NBUF,
            [pltpu.VMEM((CHUNK, D), jnp.float32)] * NBUF,
            [pltpu.SemaphoreType.DMA] * NBUF,
            [pltpu.SemaphoreType.DMA] * NBUF,
        ],
        compiler_params=pltpu.CompilerParams(use_tc_tiling_on_sc=False),
    )
    def gather(idx_hbm, table_hbm, out_hbm, idx_vs, rows_vs, gsems, osems):
        wid = lax.axis_index("s") * NC + lax.axis_index("c")
        base = wid * b_per_w

        gcopy = {}
        ocopy = {}

        def start(g):
            b = g % NBUF
            off = base + g * CHUNK
            pltpu.sync_copy(idx_hbm.at[pl.ds(off, CHUNK)], idx_vs[b])
            c = pltpu.make_async_copy(
                table_hbm.at[idx_vs[b]], rows_vs[b], gsems[b])
            c.start()
            gcopy[g] = c

        def drain(g):
            b = g % NBUF
            off = base + g * CHUNK
            gcopy[g].wait()
            c = pltpu.make_async_copy(
                rows_vs[b], out_hbm.at[pl.ds(off, CHUNK)], osems[b])
            c.start()
            ocopy[g] = c

        for g in range(nchunks):
            if g >= NBUF:
                ocopy[g - NBUF].wait()
            start(g)
            if g >= 1:
                drain(g - 1)
        drain(nchunks - 1)
        ocopy[nchunks - 2].wait()
        ocopy[nchunks - 1].wait()

    return gather


def kernel(indices, weight):
    BT, T = indices.shape
    B = BT * T
    V = weight.shape[0]
    n_full = V // 128
    rem = V - n_full * 128
    tail = weight[n_full * 128:].T.reshape(-1) if rem else jnp.zeros(
        (D,), jnp.float32)
    w2 = _detile_call(V)(weight.T, tail)
    idx = indices.T.reshape(-1).astype(jnp.int32)
    out = _gather_call(B, V)(idx, w2.reshape(V, D))
    return out.reshape(T, BT, D).transpose(1, 0, 2)
